# Initial kernel scaffold; baseline (speedup 1.0000x reference)
import functools

import jax
import jax.numpy as jnp
from jax import lax
from jax.experimental import pallas as pl
from jax.experimental.pallas import tpu as pltpu
from jax.experimental.pallas import tpu_sc as plsc

N = 10000
E = 320000
D = 128
NC = 2
NS = 16
EPT = E // (NC * NS)
C = 80
NCHUNK = EPT // C
RPT = N // NS
R = 1000

f32 = jnp.float32


def _mesh():
    return plsc.VectorSubcoreMesh(
        core_axis_name="c", subcore_axis_name="s", num_cores=NC, num_subcores=NS
    )


def _sc_count(dst_r):
    @functools.partial(
        pl.kernel,
        out_type=jax.ShapeDtypeStruct((NC, N, 16), f32),
        mesh=_mesh(),
        scratch_types=[
            pltpu.VMEM((NCHUNK, C), jnp.int32),
            pltpu.VMEM((RPT, 16), f32),
            pltpu.VMEM_SHARED((N, 16), f32),
        ],
    )
    def kcount(dst_hbm, out_hbm, dstbuf, obuf, acc):
        c = lax.axis_index("c")
        s = lax.axis_index("s")
        pltpu.sync_copy(dst_hbm.at[c, s], dstbuf)
        for i in range(RPT):
            obuf[i, :] = jnp.ones((16,), f32)
        pltpu.sync_copy(obuf, acc.at[pl.ds(s * RPT, RPT)])
        plsc.subcore_barrier()

        def body(j, carry):
            pltpu.sync_copy(obuf.at[pl.ds(0, C)], acc.at[dstbuf.at[j]], add=True)
            return carry

        lax.fori_loop(0, NCHUNK, body, 0)
        plsc.subcore_barrier()
        pltpu.sync_copy(acc.at[pl.ds(s * RPT, RPT)], out_hbm.at[c].at[pl.ds(s * RPT, RPT)])

    return kcount(dst_r)


def _sc_agg(xws, src_r, dst_r):
    @functools.partial(
        pl.kernel,
        out_type=jax.ShapeDtypeStruct((NC, N, D), f32),
        mesh=_mesh(),
        scratch_types=[
            pltpu.VMEM((NCHUNK, C), jnp.int32),
            pltpu.VMEM((NCHUNK, C), jnp.int32),
            pltpu.VMEM((C, D), f32),
            pltpu.VMEM((C, D), f32),
            pltpu.SemaphoreType.DMA,
            pltpu.SemaphoreType.DMA,
            pltpu.VMEM_SHARED((N, D), f32),
        ],
    )
    def kagg(xws_hbm, src_hbm, dst_hbm, out_hbm, srcbuf, dstbuf, row0, row1, g0, g1, acc):
        c = lax.axis_index("c")
        s = lax.axis_index("s")
        pltpu.sync_copy(src_hbm.at[c, s], srcbuf)
        pltpu.sync_copy(dst_hbm.at[c, s], dstbuf)
        pltpu.sync_copy(xws_hbm.at[pl.ds(s * RPT, RPT)], acc.at[pl.ds(s * RPT, RPT)])
        plsc.subcore_barrier()

        rows = (row0, row1)
        sems = (g0, g1)
        pltpu.async_copy(xws_hbm.at[srcbuf.at[0]], row0, g0)
        pltpu.async_copy(xws_hbm.at[srcbuf.at[1]], row1, g1)

        def body(g, carry):
            for b in range(2):
                j = g * 2 + b

                @pl.when(j < NCHUNK)
                def _():
                    pltpu.make_async_copy(xws_hbm.at[srcbuf.at[j]], rows[b], sems[b]).wait()
                    pltpu.sync_copy(rows[b], acc.at[dstbuf.at[j]], add=True)

                    @pl.when(j + 2 < NCHUNK)
                    def _():
                        pltpu.async_copy(xws_hbm.at[srcbuf.at[j + 2]], rows[b], sems[b])

            return carry

        lax.fori_loop(0, (NCHUNK + 1) // 2, body, 0)
        plsc.subcore_barrier()
        pltpu.sync_copy(acc.at[pl.ds(s * RPT, RPT)], out_hbm.at[c].at[pl.ds(s * RPT, RPT)])

    return kagg(xws, src_r, dst_r)


def _tc_mm1(x, W1, counts):
    def body(x_ref, w_ref, cnt_ref, xws_ref, dis_ref):
        dis = lax.rsqrt(cnt_ref[0] + cnt_ref[1] - 1.0)
        dis_ref[...] = dis
        t = jnp.dot(x_ref[...], w_ref[...], preferred_element_type=f32)
        xws_ref[...] = t * dis[:, 0:1]

    return pl.pallas_call(
        body,
        grid=(N // R,),
        in_specs=[
            pl.BlockSpec((R, D), lambda i: (i, 0)),
            pl.BlockSpec((D, D), lambda i: (0, 0)),
            pl.BlockSpec((NC, R, 16), lambda i: (0, i, 0)),
        ],
        out_specs=[
            pl.BlockSpec((R, D), lambda i: (i, 0)),
            pl.BlockSpec((R, 16), lambda i: (i, 0)),
        ],
        out_shape=[
            jax.ShapeDtypeStruct((N, D), f32),
            jax.ShapeDtypeStruct((N, 16), f32),
        ],
    )(x, W1, counts)


def _tc_mid(acc, xws, dis16, b, Wn):
    def body(acc_ref, xws_ref, dis_ref, b_ref, w_ref, out_ref):
        dis = dis_ref[:, 0:1]
        h = dis * (acc_ref[0] + acc_ref[1] - xws_ref[...]) + b_ref[...]
        h = jnp.maximum(h, 0.0)
        out_ref[...] = jnp.dot(h, w_ref[...], preferred_element_type=f32) * dis

    return pl.pallas_call(
        body,
        grid=(N // R,),
        in_specs=[
            pl.BlockSpec((NC, R, D), lambda i: (0, i, 0)),
            pl.BlockSpec((R, D), lambda i: (i, 0)),
            pl.BlockSpec((R, 16), lambda i: (i, 0)),
            pl.BlockSpec((1, D), lambda i: (0, 0)),
            pl.BlockSpec((D, D), lambda i: (0, 0)),
        ],
        out_specs=pl.BlockSpec((R, D), lambda i: (i, 0)),
        out_shape=jax.ShapeDtypeStruct((N, D), f32),
    )(acc, xws, dis16, b.reshape(1, D), Wn)


def _tc_final(acc, xws, dis16, b):
    def body(acc_ref, xws_ref, dis_ref, b_ref, out_ref):
        dis = dis_ref[:, 0:1]
        out_ref[...] = dis * (acc_ref[0] + acc_ref[1] - xws_ref[...]) + b_ref[...]

    return pl.pallas_call(
        body,
        grid=(N // R,),
        in_specs=[
            pl.BlockSpec((NC, R, D), lambda i: (0, i, 0)),
            pl.BlockSpec((R, D), lambda i: (i, 0)),
            pl.BlockSpec((R, 16), lambda i: (i, 0)),
            pl.BlockSpec((1, D), lambda i: (0, 0)),
        ],
        out_specs=pl.BlockSpec((R, D), lambda i: (i, 0)),
        out_shape=jax.ShapeDtypeStruct((N, D), f32),
    )(acc, xws, dis16, b.reshape(1, D))


def kernel(x, edge_index, W1, b1, W2, b2, W_out, b_out):
    src_r = edge_index[0].reshape(NC, NS, NCHUNK, C)
    dst_r = edge_index[1].reshape(NC, NS, NCHUNK, C)
    counts = _sc_count(dst_r)
    xws1, dis16 = _tc_mm1(x, W1, counts)
    acc1 = _sc_agg(xws1, src_r, dst_r)
    xws2 = _tc_mid(acc1, xws1, dis16, b1, W2)
    acc2 = _sc_agg(xws2, src_r, dst_r)
    xws3 = _tc_mid(acc2, xws2, dis16, b2, W_out)
    acc3 = _sc_agg(xws3, src_r, dst_r)
    return _tc_final(acc3, xws3, dis16, b_out)


# speedup vs baseline: 19.1061x; 19.1061x over previous
import functools

import jax
import jax.numpy as jnp
from jax import lax
from jax.experimental import pallas as pl
from jax.experimental.pallas import tpu as pltpu
from jax.experimental.pallas import tpu_sc as plsc

N = 10000
E = 320000
D = 128
NC = 2
NS = 16
EPT = E // (NC * NS)
C = 80
NCHUNK = EPT // C
RCHUNK = 640
NRK = 8
R = 1000

f32 = jnp.float32


def _mesh():
    return plsc.VectorSubcoreMesh(
        core_axis_name="c", subcore_axis_name="s", num_cores=NC, num_subcores=NS
    )


def _sc_count(dst_flat, idn):
    @functools.partial(
        pl.kernel,
        out_type=jax.ShapeDtypeStruct((NC, N, 16), f32),
        mesh=_mesh(),
        scratch_types=[
            pltpu.VMEM((C,), jnp.int32),
            pltpu.VMEM((C,), jnp.int32),
            pltpu.VMEM((C, 16), f32),
            pltpu.VMEM((C, 16), f32),
            pltpu.VMEM_SHARED((N, 16), f32),
        ],
    )
    def kcount(dst_hbm, idn_hbm, out_hbm, idbuf, dstbuf, obuf, rbuf, acc):
        c = lax.axis_index("c")
        s = lax.axis_index("s")
        ebase = (c * NS + s) * EPT
        rbase = s * RCHUNK
        for i in range(C):
            obuf[i, :] = jnp.ones((16,), f32)
        for k in range(NRK):
            off = rbase + k * C

            @pl.when(off < N)
            def _():
                pltpu.sync_copy(idn_hbm.at[pl.ds(off, C)], idbuf)
                pltpu.sync_copy(obuf, acc.at[idbuf])

        plsc.subcore_barrier()

        def body(j, carry):
            pltpu.sync_copy(dst_hbm.at[pl.ds(ebase + j * C, C)], dstbuf)
            pltpu.sync_copy(obuf, acc.at[dstbuf], add=True)
            return carry

        lax.fori_loop(0, NCHUNK, body, 0)
        plsc.subcore_barrier()
        for k in range(NRK):
            off = rbase + k * C

            @pl.when(off < N)
            def _():
                pltpu.sync_copy(idn_hbm.at[pl.ds(off, C)], idbuf)
                pltpu.sync_copy(acc.at[idbuf], rbuf)
                pltpu.sync_copy(rbuf, out_hbm.at[c].at[pl.ds(off, C)])

    return kcount(dst_flat, idn)


def _sc_agg(xws, src_flat, dst_flat, idn):
    @functools.partial(
        pl.kernel,
        out_type=jax.ShapeDtypeStruct((NC, N, D), f32),
        mesh=_mesh(),
        scratch_types=[
            pltpu.VMEM((EPT,), jnp.int32),
            pltpu.VMEM((C,), jnp.int32),
            pltpu.VMEM((C,), jnp.int32),
            pltpu.VMEM((C,), jnp.int32),
            pltpu.VMEM((C, D), f32),
            pltpu.VMEM((C, D), f32),
            pltpu.SemaphoreType.DMA,
            pltpu.SemaphoreType.DMA,
            pltpu.VMEM_SHARED((N, D), f32),
        ],
    )
    def kagg(xws_hbm, src_hbm, dst_hbm, idn_hbm, out_hbm,
             srcbuf, dst0, dst1, idbuf, row0, row1, g0, g1, acc):
        c = lax.axis_index("c")
        s = lax.axis_index("s")
        ebase = (c * NS + s) * EPT
        rbase = s * RCHUNK
        pltpu.sync_copy(src_hbm.at[pl.ds(ebase, EPT)], srcbuf)
        for k in range(NRK):
            off = rbase + k * C

            @pl.when(off < N)
            def _():
                pltpu.sync_copy(idn_hbm.at[pl.ds(off, C)], idbuf)
                pltpu.sync_copy(xws_hbm.at[pl.ds(off, C)], row0)
                pltpu.sync_copy(row0, acc.at[idbuf])

        plsc.subcore_barrier()

        dsts = (dst0, dst1)
        rows = (row0, row1)
        sems = (g0, g1)

        def gather(j, b):
            return pltpu.make_async_copy(
                xws_hbm.at[srcbuf.at[pl.ds(j * C, C)]], rows[b], sems[b]
            )

        pltpu.sync_copy(dst_hbm.at[pl.ds(ebase, C)], dst0)
        pltpu.sync_copy(dst_hbm.at[pl.ds(ebase + C, C)], dst1)
        gather(0, 0).start()
        gather(1, 1).start()

        def body(g, carry):
            for b in range(2):
                j = 2 * g + b

                @pl.when(j < NCHUNK)
                def _():
                    gather(j, b).wait()
                    pltpu.sync_copy(rows[b], acc.at[dsts[b]], add=True)

                    @pl.when(j + 2 < NCHUNK)
                    def _():
                        pltpu.sync_copy(
                            dst_hbm.at[pl.ds(ebase + (j + 2) * C, C)], dsts[b]
                        )
                        gather(j + 2, b).start()

            return carry

        lax.fori_loop(0, (NCHUNK + 1) // 2, body, 0)
        plsc.subcore_barrier()
        for k in range(NRK):
            off = rbase + k * C

            @pl.when(off < N)
            def _():
                pltpu.sync_copy(idn_hbm.at[pl.ds(off, C)], idbuf)
                pltpu.sync_copy(acc.at[idbuf], row0)
                pltpu.sync_copy(row0, out_hbm.at[c].at[pl.ds(off, C)])

    return kagg(xws, src_flat, dst_flat, idn)


def _tc_mm1(x, W1, counts):
    def body(x_ref, w_ref, cnt_ref, xws_ref, dis_ref):
        dis = lax.rsqrt(cnt_ref[0] + cnt_ref[1] - 1.0)
        dis_ref[...] = dis
        t = jnp.dot(x_ref[...], w_ref[...], preferred_element_type=f32)
        xws_ref[...] = t * dis[:, 0:1]

    return pl.pallas_call(
        body,
        grid=(N // R,),
        in_specs=[
            pl.BlockSpec((R, D), lambda i: (i, 0)),
            pl.BlockSpec((D, D), lambda i: (0, 0)),
            pl.BlockSpec((NC, R, 16), lambda i: (0, i, 0)),
        ],
        out_specs=[
            pl.BlockSpec((R, D), lambda i: (i, 0)),
            pl.BlockSpec((R, 16), lambda i: (i, 0)),
        ],
        out_shape=[
            jax.ShapeDtypeStruct((N, D), f32),
            jax.ShapeDtypeStruct((N, 16), f32),
        ],
    )(x, W1, counts)


def _tc_mid(acc, xws, dis16, b, Wn):
    def body(acc_ref, xws_ref, dis_ref, b_ref, w_ref, out_ref):
        dis = dis_ref[:, 0:1]
        h = dis * (acc_ref[0] + acc_ref[1] - xws_ref[...]) + b_ref[...]
        h = jnp.maximum(h, 0.0)
        out_ref[...] = jnp.dot(h, w_ref[...], preferred_element_type=f32) * dis

    return pl.pallas_call(
        body,
        grid=(N // R,),
        in_specs=[
            pl.BlockSpec((NC, R, D), lambda i: (0, i, 0)),
            pl.BlockSpec((R, D), lambda i: (i, 0)),
            pl.BlockSpec((R, 16), lambda i: (i, 0)),
            pl.BlockSpec((1, D), lambda i: (0, 0)),
            pl.BlockSpec((D, D), lambda i: (0, 0)),
        ],
        out_specs=pl.BlockSpec((R, D), lambda i: (i, 0)),
        out_shape=jax.ShapeDtypeStruct((N, D), f32),
    )(acc, xws, dis16, b.reshape(1, D), Wn)


def _tc_final(acc, xws, dis16, b):
    def body(acc_ref, xws_ref, dis_ref, b_ref, out_ref):
        dis = dis_ref[:, 0:1]
        out_ref[...] = dis * (acc_ref[0] + acc_ref[1] - xws_ref[...]) + b_ref[...]

    return pl.pallas_call(
        body,
        grid=(N // R,),
        in_specs=[
            pl.BlockSpec((NC, R, D), lambda i: (0, i, 0)),
            pl.BlockSpec((R, D), lambda i: (i, 0)),
            pl.BlockSpec((R, 16), lambda i: (i, 0)),
            pl.BlockSpec((1, D), lambda i: (0, 0)),
        ],
        out_specs=pl.BlockSpec((R, D), lambda i: (i, 0)),
        out_shape=jax.ShapeDtypeStruct((N, D), f32),
    )(acc, xws, dis16, b.reshape(1, D))


def kernel(x, edge_index, W1, b1, W2, b2, W_out, b_out):
    src_flat = edge_index[0].reshape(E)
    dst_flat = edge_index[1].reshape(E)
    idn = jnp.arange(N, dtype=jnp.int32)
    counts = _sc_count(dst_flat, idn)
    xws1, dis16 = _tc_mm1(x, W1, counts)
    acc1 = _sc_agg(xws1, src_flat, dst_flat, idn)
    xws2 = _tc_mid(acc1, xws1, dis16, b1, W2)
    acc2 = _sc_agg(xws2, src_flat, dst_flat, idn)
    xws3 = _tc_mid(acc2, xws2, dis16, b2, W_out)
    acc3 = _sc_agg(xws3, src_flat, dst_flat, idn)
    return _tc_final(acc3, xws3, dis16, b_out)


# speedup vs baseline: 23.8341x; 1.2475x over previous
import functools

import jax
import jax.numpy as jnp
from jax import lax
from jax.experimental import pallas as pl
from jax.experimental.pallas import tpu as pltpu
from jax.experimental.pallas import tpu_sc as plsc

N = 10000
E = 320000
D = 128
NC = 2
NS = 16
EPT = E // (NC * NS)
C = 80
NCHUNK = EPT // C
RCHUNK = 640
NRK = 8
R = 1000

f32 = jnp.float32


def _mesh():
    return plsc.VectorSubcoreMesh(
        core_axis_name="c", subcore_axis_name="s", num_cores=NC, num_subcores=NS
    )


def _sc_count(dst_flat, idn):
    @functools.partial(
        pl.kernel,
        out_type=jax.ShapeDtypeStruct((NC, N, 16), f32),
        mesh=_mesh(),
        scratch_types=[
            pltpu.VMEM((C,), jnp.int32),
            pltpu.VMEM((NCHUNK, C), jnp.int32),
            pltpu.VMEM((C, 16), f32),
            pltpu.VMEM((C, 16), f32),
            pltpu.VMEM_SHARED((N, 16), f32),
        ],
    )
    def kcount(dst_hbm, idn_hbm, out_hbm, idbuf, dstbuf, obuf, rbuf, acc):
        c = lax.axis_index("c")
        s = lax.axis_index("s")
        rbase = s * RCHUNK
        for i in range(C):
            obuf[i, :] = jnp.ones((16,), f32)
        for k in range(NRK):
            off = rbase + k * C

            @pl.when(off < N)
            def _():
                pltpu.sync_copy(idn_hbm.at[pl.ds(off, C)], idbuf)
                pltpu.sync_copy(obuf, acc.at[idbuf])

        pltpu.sync_copy(dst_hbm.at[c, s], dstbuf)
        plsc.subcore_barrier()

        def body(j, carry):
            pltpu.sync_copy(obuf, acc.at[dstbuf.at[j]], add=True)
            return carry

        lax.fori_loop(0, NCHUNK, body, 0)
        plsc.subcore_barrier()
        for k in range(NRK):
            off = rbase + k * C

            @pl.when(off < N)
            def _():
                pltpu.sync_copy(idn_hbm.at[pl.ds(off, C)], idbuf)
                pltpu.sync_copy(acc.at[idbuf], rbuf)
                pltpu.sync_copy(rbuf, out_hbm.at[c].at[pl.ds(off, C)])

    return kcount(dst_flat, idn)


def _sc_agg(xws, src_flat, dst_flat, idn):
    @functools.partial(
        pl.kernel,
        out_type=jax.ShapeDtypeStruct((NC, N, D), f32),
        mesh=_mesh(),
        scratch_types=[
            pltpu.VMEM((EPT,), jnp.int32),
            pltpu.VMEM((NCHUNK, C), jnp.int32),
            pltpu.VMEM((C,), jnp.int32),
            pltpu.VMEM((C, D), f32),
            pltpu.VMEM((C, D), f32),
            pltpu.SemaphoreType.DMA,
            pltpu.SemaphoreType.DMA,
            pltpu.VMEM_SHARED((N, D), f32),
        ],
    )
    def kagg(xws_hbm, src_hbm, dst_hbm, idn_hbm, out_hbm,
             srcbuf, dstbuf, idbuf, row0, row1, g0, g1, acc):
        c = lax.axis_index("c")
        s = lax.axis_index("s")
        ebase = (c * NS + s) * EPT
        rbase = s * RCHUNK
        pltpu.sync_copy(src_hbm.at[pl.ds(ebase, EPT)], srcbuf)
        pltpu.sync_copy(dst_hbm.at[c, s], dstbuf)
        for k in range(NRK):
            off = rbase + k * C

            @pl.when(off < N)
            def _():
                pltpu.sync_copy(idn_hbm.at[pl.ds(off, C)], idbuf)
                pltpu.sync_copy(xws_hbm.at[pl.ds(off, C)], row0)
                pltpu.sync_copy(row0, acc.at[idbuf])

        plsc.subcore_barrier()

        rows = (row0, row1)
        sems = (g0, g1)
        NB = 2

        def gather(j, b):
            return pltpu.make_async_copy(
                xws_hbm.at[srcbuf.at[pl.ds(j * C, C)]], rows[b], sems[b]
            )

        for b in range(NB):
            gather(b, b).start()

        def body(g, carry):
            for b in range(NB):
                j = NB * g + b

                @pl.when(j < NCHUNK)
                def _():
                    gather(j, b).wait()
                    pltpu.sync_copy(rows[b], acc.at[dstbuf.at[j]], add=True)

                    @pl.when(j + NB < NCHUNK)
                    def _():
                        gather(j + NB, b).start()

            return carry

        lax.fori_loop(0, (NCHUNK + NB - 1) // NB, body, 0)
        plsc.subcore_barrier()
        for k in range(NRK):
            off = rbase + k * C

            @pl.when(off < N)
            def _():
                pltpu.sync_copy(idn_hbm.at[pl.ds(off, C)], idbuf)
                pltpu.sync_copy(acc.at[idbuf], row0)
                pltpu.sync_copy(row0, out_hbm.at[c].at[pl.ds(off, C)])

    return kagg(xws, src_flat, dst_flat, idn)


def _tc_mm1(x, W1, counts):
    def body(x_ref, w_ref, cnt_ref, xws_ref, dis_ref):
        dis = lax.rsqrt(cnt_ref[0] + cnt_ref[1] - 1.0)
        dis_ref[...] = dis
        t = jnp.dot(x_ref[...], w_ref[...], preferred_element_type=f32)
        xws_ref[...] = t * dis[:, 0:1]

    return pl.pallas_call(
        body,
        grid=(N // R,),
        in_specs=[
            pl.BlockSpec((R, D), lambda i: (i, 0)),
            pl.BlockSpec((D, D), lambda i: (0, 0)),
            pl.BlockSpec((NC, R, 16), lambda i: (0, i, 0)),
        ],
        out_specs=[
            pl.BlockSpec((R, D), lambda i: (i, 0)),
            pl.BlockSpec((R, 16), lambda i: (i, 0)),
        ],
        out_shape=[
            jax.ShapeDtypeStruct((N, D), f32),
            jax.ShapeDtypeStruct((N, 16), f32),
        ],
    )(x, W1, counts)


def _tc_mid(acc, xws, dis16, b, Wn):
    def body(acc_ref, xws_ref, dis_ref, b_ref, w_ref, out_ref):
        dis = dis_ref[:, 0:1]
        h = dis * (acc_ref[0] + acc_ref[1] - xws_ref[...]) + b_ref[...]
        h = jnp.maximum(h, 0.0)
        out_ref[...] = jnp.dot(h, w_ref[...], preferred_element_type=f32) * dis

    return pl.pallas_call(
        body,
        grid=(N // R,),
        in_specs=[
            pl.BlockSpec((NC, R, D), lambda i: (0, i, 0)),
            pl.BlockSpec((R, D), lambda i: (i, 0)),
            pl.BlockSpec((R, 16), lambda i: (i, 0)),
            pl.BlockSpec((1, D), lambda i: (0, 0)),
            pl.BlockSpec((D, D), lambda i: (0, 0)),
        ],
        out_specs=pl.BlockSpec((R, D), lambda i: (i, 0)),
        out_shape=jax.ShapeDtypeStruct((N, D), f32),
    )(acc, xws, dis16, b.reshape(1, D), Wn)


def _tc_final(acc, xws, dis16, b):
    def body(acc_ref, xws_ref, dis_ref, b_ref, out_ref):
        dis = dis_ref[:, 0:1]
        out_ref[...] = dis * (acc_ref[0] + acc_ref[1] - xws_ref[...]) + b_ref[...]

    return pl.pallas_call(
        body,
        grid=(N // R,),
        in_specs=[
            pl.BlockSpec((NC, R, D), lambda i: (0, i, 0)),
            pl.BlockSpec((R, D), lambda i: (i, 0)),
            pl.BlockSpec((R, 16), lambda i: (i, 0)),
            pl.BlockSpec((1, D), lambda i: (0, 0)),
        ],
        out_specs=pl.BlockSpec((R, D), lambda i: (i, 0)),
        out_shape=jax.ShapeDtypeStruct((N, D), f32),
    )(acc, xws, dis16, b.reshape(1, D))


def kernel(x, edge_index, W1, b1, W2, b2, W_out, b_out):
    src_flat = edge_index[0].reshape(E)
    dst_r3 = edge_index[1].reshape(NC, NS, NCHUNK, C)
    idn = jnp.arange(N, dtype=jnp.int32)
    counts = _sc_count(dst_r3, idn)
    xws1, dis16 = _tc_mm1(x, W1, counts)
    acc1 = _sc_agg(xws1, src_flat, dst_r3, idn)
    xws2 = _tc_mid(acc1, xws1, dis16, b1, W2)
    acc2 = _sc_agg(xws2, src_flat, dst_r3, idn)
    xws3 = _tc_mid(acc2, xws2, dis16, b2, W_out)
    acc3 = _sc_agg(xws3, src_flat, dst_r3, idn)
    return _tc_final(acc3, xws3, dis16, b_out)


# speedup vs baseline: 27.6574x; 1.1604x over previous
import functools

import jax
import jax.numpy as jnp
from jax import lax
from jax.experimental import pallas as pl
from jax.experimental.pallas import tpu as pltpu
from jax.experimental.pallas import tpu_sc as plsc

N = 10000
E = 320000
D = 128
NC = 2
NS = 16
EPT = E // (NC * NS)
C = 80
NCHUNK = EPT // C
RCHUNK = 640
NRK = 8
R = 1000

f32 = jnp.float32


def _mesh():
    return plsc.VectorSubcoreMesh(
        core_axis_name="c", subcore_axis_name="s", num_cores=NC, num_subcores=NS
    )


def _sc_count(dst_flat, idn):
    @functools.partial(
        pl.kernel,
        out_type=jax.ShapeDtypeStruct((NC, N, 16), f32),
        mesh=_mesh(),
        scratch_types=[
            pltpu.VMEM((C,), jnp.int32),
            pltpu.VMEM((NCHUNK, C), jnp.int32),
            pltpu.VMEM((C, 16), f32),
            pltpu.VMEM((C, 16), f32),
            pltpu.VMEM_SHARED((N, 16), f32),
        ],
    )
    def kcount(dst_hbm, idn_hbm, out_hbm, idbuf, dstbuf, obuf, rbuf, acc):
        c = lax.axis_index("c")
        s = lax.axis_index("s")
        rbase = s * RCHUNK
        for i in range(C):
            obuf[i, :] = jnp.ones((16,), f32)
        for k in range(NRK):
            off = rbase + k * C

            @pl.when(off < N)
            def _():
                pltpu.sync_copy(idn_hbm.at[pl.ds(off, C)], idbuf)
                pltpu.sync_copy(obuf, acc.at[idbuf])

        pltpu.sync_copy(dst_hbm.at[c, s], dstbuf)
        plsc.subcore_barrier()

        def body(j, carry):
            pltpu.sync_copy(obuf, acc.at[dstbuf.at[j]], add=True)
            return carry

        lax.fori_loop(0, NCHUNK, body, 0)
        plsc.subcore_barrier()
        for k in range(NRK):
            off = rbase + k * C

            @pl.when(off < N)
            def _():
                pltpu.sync_copy(idn_hbm.at[pl.ds(off, C)], idbuf)
                pltpu.sync_copy(acc.at[idbuf], rbuf)
                pltpu.sync_copy(rbuf, out_hbm.at[c].at[pl.ds(off, C)])

    return kcount(dst_flat, idn)


def _sc_agg(xws, src_flat, dst_flat, idn):
    @functools.partial(
        pl.kernel,
        out_type=jax.ShapeDtypeStruct((NC, N, D), f32),
        mesh=_mesh(),
        scratch_types=[
            pltpu.VMEM((EPT,), jnp.int32),
            pltpu.VMEM((C,), jnp.int32),
            pltpu.VMEM((C,), jnp.int32),
            pltpu.VMEM((C,), jnp.int32),
            pltpu.VMEM((C,), jnp.int32),
            pltpu.VMEM((C, D), f32),
            pltpu.VMEM((C, D), f32),
            pltpu.VMEM((C, D), f32),
            pltpu.SemaphoreType.DMA,
            pltpu.SemaphoreType.DMA,
            pltpu.SemaphoreType.DMA,
            pltpu.SemaphoreType.DMA,
            pltpu.SemaphoreType.DMA,
            pltpu.SemaphoreType.DMA,
            pltpu.VMEM_SHARED((N, D), f32),
        ],
    )
    def kagg(xws_hbm, src_hbm, dst_hbm, idn_hbm, out_hbm,
             srcbuf, dst0, dst1, dst2, idbuf, row0, row1, row2,
             g0, g1, g2, d0, d1, d2, acc):
        c = lax.axis_index("c")
        s = lax.axis_index("s")
        ebase = (c * NS + s) * EPT
        rbase = s * RCHUNK
        pltpu.sync_copy(src_hbm.at[pl.ds(ebase, EPT)], srcbuf)
        for k in range(NRK):
            off = rbase + k * C

            @pl.when(off < N)
            def _():
                pltpu.sync_copy(idn_hbm.at[pl.ds(off, C)], idbuf)
                pltpu.sync_copy(xws_hbm.at[pl.ds(off, C)], row0)
                pltpu.sync_copy(row0, acc.at[idbuf])

        plsc.subcore_barrier()

        rows = (row0, row1, row2)
        gsems = (g0, g1, g2)
        dsts = (dst0, dst1, dst2)
        dsems = (d0, d1, d2)
        NB = 3

        def gather(j, b):
            return pltpu.make_async_copy(
                xws_hbm.at[srcbuf.at[pl.ds(j * C, C)]], rows[b], gsems[b]
            )

        def dstage(j, b):
            return pltpu.make_async_copy(
                dst_hbm.at[pl.ds(ebase + j * C, C)], dsts[b], dsems[b]
            )

        for b in range(NB):
            dstage(b, b).start()
            gather(b, b).start()

        def body(g, carry):
            for b in range(NB):
                j = NB * g + b

                @pl.when(j < NCHUNK)
                def _():
                    dstage(j, b).wait()
                    gather(j, b).wait()
                    pltpu.sync_copy(rows[b], acc.at[dsts[b]], add=True)

                    @pl.when(j + NB < NCHUNK)
                    def _():
                        dstage(j + NB, b).start()
                        gather(j + NB, b).start()

            return carry

        lax.fori_loop(0, (NCHUNK + NB - 1) // NB, body, 0)
        plsc.subcore_barrier()
        for k in range(NRK):
            off = rbase + k * C

            @pl.when(off < N)
            def _():
                pltpu.sync_copy(idn_hbm.at[pl.ds(off, C)], idbuf)
                pltpu.sync_copy(acc.at[idbuf], row0)
                pltpu.sync_copy(row0, out_hbm.at[c].at[pl.ds(off, C)])

    return kagg(xws, src_flat, dst_flat, idn)


def _tc_mm1(x, W1, counts):
    def body(x_ref, w_ref, cnt_ref, xws_ref, dis_ref):
        dis = lax.rsqrt(cnt_ref[0] + cnt_ref[1] - 1.0)
        dis_ref[...] = dis
        t = jnp.dot(x_ref[...], w_ref[...], preferred_element_type=f32)
        xws_ref[...] = t * dis[:, 0:1]

    return pl.pallas_call(
        body,
        grid=(N // R,),
        in_specs=[
            pl.BlockSpec((R, D), lambda i: (i, 0)),
            pl.BlockSpec((D, D), lambda i: (0, 0)),
            pl.BlockSpec((NC, R, 16), lambda i: (0, i, 0)),
        ],
        out_specs=[
            pl.BlockSpec((R, D), lambda i: (i, 0)),
            pl.BlockSpec((R, 16), lambda i: (i, 0)),
        ],
        out_shape=[
            jax.ShapeDtypeStruct((N, D), f32),
            jax.ShapeDtypeStruct((N, 16), f32),
        ],
    )(x, W1, counts)


def _tc_mid(acc, xws, dis16, b, Wn):
    def body(acc_ref, xws_ref, dis_ref, b_ref, w_ref, out_ref):
        dis = dis_ref[:, 0:1]
        h = dis * (acc_ref[0] + acc_ref[1] - xws_ref[...]) + b_ref[...]
        h = jnp.maximum(h, 0.0)
        out_ref[...] = jnp.dot(h, w_ref[...], preferred_element_type=f32) * dis

    return pl.pallas_call(
        body,
        grid=(N // R,),
        in_specs=[
            pl.BlockSpec((NC, R, D), lambda i: (0, i, 0)),
            pl.BlockSpec((R, D), lambda i: (i, 0)),
            pl.BlockSpec((R, 16), lambda i: (i, 0)),
            pl.BlockSpec((1, D), lambda i: (0, 0)),
            pl.BlockSpec((D, D), lambda i: (0, 0)),
        ],
        out_specs=pl.BlockSpec((R, D), lambda i: (i, 0)),
        out_shape=jax.ShapeDtypeStruct((N, D), f32),
    )(acc, xws, dis16, b.reshape(1, D), Wn)


def _tc_final(acc, xws, dis16, b):
    def body(acc_ref, xws_ref, dis_ref, b_ref, out_ref):
        dis = dis_ref[:, 0:1]
        out_ref[...] = dis * (acc_ref[0] + acc_ref[1] - xws_ref[...]) + b_ref[...]

    return pl.pallas_call(
        body,
        grid=(N // R,),
        in_specs=[
            pl.BlockSpec((NC, R, D), lambda i: (0, i, 0)),
            pl.BlockSpec((R, D), lambda i: (i, 0)),
            pl.BlockSpec((R, 16), lambda i: (i, 0)),
            pl.BlockSpec((1, D), lambda i: (0, 0)),
        ],
        out_specs=pl.BlockSpec((R, D), lambda i: (i, 0)),
        out_shape=jax.ShapeDtypeStruct((N, D), f32),
    )(acc, xws, dis16, b.reshape(1, D))


def kernel(x, edge_index, W1, b1, W2, b2, W_out, b_out):
    src_flat = edge_index[0].reshape(E)
    dst_flat = edge_index[1].reshape(E)
    dst_r3 = edge_index[1].reshape(NC, NS, NCHUNK, C)
    idn = jnp.arange(N, dtype=jnp.int32)
    counts = _sc_count(dst_r3, idn)
    xws1, dis16 = _tc_mm1(x, W1, counts)
    acc1 = _sc_agg(xws1, src_flat, dst_flat, idn)
    xws2 = _tc_mid(acc1, xws1, dis16, b1, W2)
    acc2 = _sc_agg(xws2, src_flat, dst_flat, idn)
    xws3 = _tc_mid(acc2, xws2, dis16, b2, W_out)
    acc3 = _sc_agg(xws3, src_flat, dst_flat, idn)
    return _tc_final(acc3, xws3, dis16, b_out)
